# baseline (speedup 1.0000x reference)
import jax
import jax.numpy as jnp
from jax.experimental import pallas as pl
from jax.experimental.pallas import tpu as pltpu

_DIM = 1024
_E = 8
_T = 2048
_F = 4 * _DIM
_CAP = 640
_MAINB = _E * _CAP
_B2 = 256
_OVG = 14
_OVR = _OVG * _B2
_GP = 64
_TB = 256

_f32 = jnp.float32
_bf16 = jnp.bfloat16


def _dotT(a, b):
    return jax.lax.dot_general(a, b, (((0,), (0,)), ((), ())),
                               preferred_element_type=_f32)


def _router_body(x_ref, wg_ref, bg_ref, pos_ref, sc_ref, ssum_ref, meta_ref):
    x = x_ref[...]
    logits = jnp.dot(x, wg_ref[...], preferred_element_type=_f32)
    logits = logits + bg_ref[...]
    m = jnp.max(logits, axis=-1, keepdims=True)
    p = jnp.exp(logits - m)
    scores = p / jnp.sum(p, axis=-1, keepdims=True)

    eio = jax.lax.broadcasted_iota(jnp.int32, (_T, _E), 1)
    m1 = jnp.max(scores, axis=-1, keepdims=True)
    i1 = jnp.min(jnp.where(scores >= m1, eio, _E), axis=-1, keepdims=True)
    masked = jnp.where(eio == i1, -jnp.inf, scores)
    m2 = jnp.max(masked, axis=-1, keepdims=True)
    i2 = jnp.min(jnp.where(masked >= m2, eio, _E), axis=-1, keepdims=True)

    a0 = (eio == i1).astype(_f32)
    a1 = (eio == i2).astype(_f32)

    ti = jax.lax.broadcasted_iota(jnp.int32, (_T, _T), 0)
    tj = jax.lax.broadcasted_iota(jnp.int32, (_T, _T), 1)
    ltri = (ti >= tj).astype(_f32)
    a01 = jnp.concatenate([a0, a1], axis=1)
    cs = jnp.dot(ltri, a01, preferred_element_type=_f32)
    cs0 = cs[:, :_E]
    cs1 = cs[:, _E:]
    tot0 = cs0[_T - 1:_T, :]
    tot1 = cs1[_T - 1:_T, :]
    counts = (tot0 + tot1).astype(jnp.int32)

    ov = jnp.maximum(counts - _CAP, 0)
    ovblk = jnp.right_shift(ov + (_B2 - 1), 8)
    ei = jax.lax.broadcasted_iota(jnp.int32, (_E, _E), 0)
    ej = jax.lax.broadcasted_iota(jnp.int32, (_E, _E), 1)
    strict = (ei < ej).astype(_f32)
    ovex = jnp.dot(ovblk.astype(_f32), strict,
                   preferred_element_type=_f32)
    ovincl = ovex + ovblk.astype(_f32)
    ovstart = float(_MAINB) + ovex * float(_B2)

    ecap = (jax.lax.broadcasted_iota(jnp.int32, (1, _E), 1)
            .astype(_f32)) * float(_CAP)

    rank0 = jnp.sum(a0 * cs0, axis=1, keepdims=True) - 1.0
    rank1 = (jnp.sum(a1 * cs1, axis=1, keepdims=True) - 1.0
             + jnp.sum(a1 * tot0, axis=1, keepdims=True))
    st0m = jnp.sum(a0 * ecap, axis=1, keepdims=True)
    st1m = jnp.sum(a1 * ecap, axis=1, keepdims=True)
    st0o = jnp.sum(a0 * ovstart, axis=1, keepdims=True)
    st1o = jnp.sum(a1 * ovstart, axis=1, keepdims=True)
    pos0 = jnp.where(rank0 < float(_CAP), st0m + rank0,
                     st0o + rank0 - float(_CAP))
    pos1 = jnp.where(rank1 < float(_CAP), st1m + rank1,
                     st1o + rank1 - float(_CAP))
    pos_ref[...] = jnp.concatenate(
        [pos0, pos1], axis=1).astype(jnp.int32)
    sc_ref[...] = jnp.concatenate([m1, m2], axis=1)
    ssum_ref[...] = m1 + m2

    gi = jax.lax.broadcasted_iota(jnp.int32, (_GP, _E), 0)
    inb = jnp.broadcast_to(ovincl.astype(jnp.int32), (_GP, _E))
    beo = jnp.minimum(
        jnp.sum(jnp.where(gi >= inb, 1, 0), axis=1, keepdims=True), _E - 1)
    tot_blk = ovincl[0:1, _E - 1:_E].astype(jnp.int32)
    gj = jax.lax.broadcasted_iota(jnp.int32, (_GP, 1), 0)
    act = jnp.where(gj < tot_blk, 1, 0)
    meta_ref[...] = jnp.concatenate([beo, act], axis=1)


def _main_body(x_ref, pos_ref, sc_ref, w1_ref, b1_ref, w2_ref,
               yw_ref, d_ref):
    e = pl.program_id(0)

    @pl.when(e == 0)
    def _dz():
        d_ref[...] = jnp.zeros_like(d_ref)

    pos = pos_ref[...]
    liota = jax.lax.broadcasted_iota(jnp.int32, (_T, _CAP), 1) + e * _CAP
    m0 = (pos[:, 0:1] == liota).astype(_bf16)
    m1 = (pos[:, 1:2] == liota).astype(_bf16)
    mt = m0 + m1

    xg = _dotT(mt, x_ref[...]).astype(_bf16)
    sc = sc_ref[...].astype(_bf16)
    w = _dotT(m0, sc[:, 0:1]) + _dotT(m1, sc[:, 1:2])

    fh = _F // 2
    y = jnp.zeros((_CAP, _DIM), _f32)
    d = jnp.zeros((1, _DIM), _f32)
    for f in range(2):
        w1c = w1_ref[0, :, f * fh:(f + 1) * fh]
        w2c = w2_ref[0, f * fh:(f + 1) * fh, :]
        b1c = b1_ref[0, 0, f * fh:(f + 1) * fh].reshape(1, fh)
        rbc = jnp.maximum(b1c, 0.0)
        hf = jnp.maximum(
            jnp.dot(xg, w1c, preferred_element_type=_f32) + b1c, 0.0) - rbc
        y = y + jnp.dot(hf.astype(_bf16), w2c, preferred_element_type=_f32)
        d = d + jnp.dot(rbc.astype(_bf16), w2c, preferred_element_type=_f32)

    d_ref[...] += d
    yw_ref[...] = (w * y).astype(_bf16)


def _ov_body(beo_ref, act_ref, x_ref, pos_ref, sc_ref, b1_ref,
             w1_ref, w2_ref, yw_ref, w1buf, w2buf, w1sem, w2sem):
    g = pl.program_id(0)
    e = beo_ref[g]
    act = act_ref[g] == 1
    gp = jnp.maximum(g - 1, 0)
    same = jnp.logical_and(act_ref[gp] == 1, beo_ref[gp] == e)
    docopy = jnp.logical_and(act, jnp.logical_or(g == 0,
                                                 jnp.logical_not(same)))

    @pl.when(docopy)
    def _cp():
        c1 = pltpu.make_async_copy(w1_ref.at[e], w1buf, w1sem)
        c2 = pltpu.make_async_copy(w2_ref.at[e], w2buf, w2sem)
        c1.start()
        c2.start()
        c1.wait()
        c2.wait()

    @pl.when(jnp.logical_not(act))
    def _zero():
        yw_ref[...] = jnp.zeros_like(yw_ref)

    @pl.when(act)
    def _compute():
        ohe = (jax.lax.broadcasted_iota(jnp.int32, (1, _E), 1)
               == e).astype(_f32)
        b1v = jnp.dot(ohe, b1_ref[...], preferred_element_type=_f32)
        rb = jnp.maximum(b1v, 0.0)
        pos = pos_ref[...]
        liota = (jax.lax.broadcasted_iota(jnp.int32, (_T, _B2), 1)
                 + _MAINB + g * _B2)
        m0 = (pos[:, 0:1] == liota).astype(_bf16)
        m1 = (pos[:, 1:2] == liota).astype(_bf16)
        mt = m0 + m1
        xg = _dotT(mt, x_ref[...]).astype(_bf16)
        sc = sc_ref[...].astype(_bf16)
        w = _dotT(m0, sc[:, 0:1]) + _dotT(m1, sc[:, 1:2])
        h = jnp.maximum(
            jnp.dot(xg, w1buf[...], preferred_element_type=_f32)
            + b1v, 0.0) - rb
        y = jnp.dot(h.astype(_bf16), w2buf[...], preferred_element_type=_f32)
        yw_ref[...] = (w * y).astype(_bf16)


def _combine_body(pos_ref, ssum_ref, d_ref, b2_ref, ywm_ref, ywo_ref,
                  out_ref):
    crow = d_ref[...] + jnp.sum(b2_ref[...], axis=0, keepdims=True)
    pos = pos_ref[...]
    ci = jax.lax.broadcasted_iota(jnp.int32, (_TB, _MAINB), 1)
    mm = ((pos[:, 0:1] == ci).astype(_bf16)
          + (pos[:, 1:2] == ci).astype(_bf16))
    co = jax.lax.broadcasted_iota(jnp.int32, (_TB, _OVR), 1) + _MAINB
    mo = ((pos[:, 0:1] == co).astype(_bf16)
          + (pos[:, 1:2] == co).astype(_bf16))
    out_ref[...] = (ssum_ref[...] * crow
                    + jnp.dot(mm, ywm_ref[...], preferred_element_type=_f32)
                    + jnp.dot(mo, ywo_ref[...], preferred_element_type=_f32))


def kernel(x, w_g, b_g, W1, b1, W2, b2):
    pos, sc, ssum, meta = pl.pallas_call(
        _router_body,
        out_shape=(
            jax.ShapeDtypeStruct((_T, 2), jnp.int32),
            jax.ShapeDtypeStruct((_T, 2), _f32),
            jax.ShapeDtypeStruct((_T, 1), _f32),
            jax.ShapeDtypeStruct((_GP, 2), jnp.int32),
        ),
        in_specs=[
            pl.BlockSpec((_T, _DIM), lambda: (0, 0)),
            pl.BlockSpec((_DIM, _E), lambda: (0, 0)),
            pl.BlockSpec((1, _E), lambda: (0, 0)),
        ],
        out_specs=(
            pl.BlockSpec((_T, 2), lambda: (0, 0)),
            pl.BlockSpec((_T, 2), lambda: (0, 0)),
            pl.BlockSpec((_T, 1), lambda: (0, 0)),
            pl.BlockSpec((_GP, 2), lambda: (0, 0)),
        ),
    )(x, w_g, b_g.reshape(1, _E))

    beo = meta[:_OVG, 0]
    act = meta[:_OVG, 1]
    xb = x.astype(_bf16)
    w1b = W1.astype(_bf16)
    w2b = W2.astype(_bf16)

    ywm, dacc = pl.pallas_call(
        _main_body,
        grid=(_E,),
        out_shape=(
            jax.ShapeDtypeStruct((_MAINB, _DIM), _bf16),
            jax.ShapeDtypeStruct((1, _DIM), _f32),
        ),
        in_specs=[
            pl.BlockSpec((_T, _DIM), lambda e: (0, 0)),
            pl.BlockSpec((_T, 2), lambda e: (0, 0)),
            pl.BlockSpec((_T, 2), lambda e: (0, 0)),
            pl.BlockSpec((1, _DIM, _F), lambda e: (e, 0, 0)),
            pl.BlockSpec((1, 1, _F), lambda e: (e, 0, 0)),
            pl.BlockSpec((1, _F, _DIM), lambda e: (e, 0, 0)),
        ],
        out_specs=(
            pl.BlockSpec((_CAP, _DIM), lambda e: (e, 0)),
            pl.BlockSpec((1, _DIM), lambda e: (0, 0)),
        ),
        compiler_params=pltpu.CompilerParams(
            dimension_semantics=("arbitrary",),
        ),
    )(xb, pos, sc, w1b, b1.reshape(_E, 1, _F), w2b)

    ov_spec = pltpu.PrefetchScalarGridSpec(
        num_scalar_prefetch=2,
        grid=(_OVG,),
        in_specs=[
            pl.BlockSpec((_T, _DIM), lambda g, beo, act: (0, 0)),
            pl.BlockSpec((_T, 2), lambda g, beo, act: (0, 0)),
            pl.BlockSpec((_T, 2), lambda g, beo, act: (0, 0)),
            pl.BlockSpec((_E, _F), lambda g, beo, act: (0, 0)),
            pl.BlockSpec(memory_space=pltpu.HBM),
            pl.BlockSpec(memory_space=pltpu.HBM),
        ],
        out_specs=pl.BlockSpec((_B2, _DIM), lambda g, beo, act: (g, 0)),
        scratch_shapes=[
            pltpu.VMEM((_DIM, _F), _bf16),
            pltpu.VMEM((_F, _DIM), _bf16),
            pltpu.SemaphoreType.DMA,
            pltpu.SemaphoreType.DMA,
        ],
    )
    ywo = pl.pallas_call(
        _ov_body,
        grid_spec=ov_spec,
        out_shape=jax.ShapeDtypeStruct((_OVR, _DIM), _bf16),
        compiler_params=pltpu.CompilerParams(
            dimension_semantics=("arbitrary",),
        ),
    )(beo, act, xb, pos, sc, b1, w1b, w2b)

    out = pl.pallas_call(
        _combine_body,
        grid=(_T // _TB,),
        out_shape=jax.ShapeDtypeStruct((_T, _DIM), _f32),
        in_specs=[
            pl.BlockSpec((_TB, 2), lambda t: (t, 0)),
            pl.BlockSpec((_TB, 1), lambda t: (t, 0)),
            pl.BlockSpec((1, _DIM), lambda t: (0, 0)),
            pl.BlockSpec((_E, _DIM), lambda t: (0, 0)),
            pl.BlockSpec((_MAINB, _DIM), lambda t: (0, 0)),
            pl.BlockSpec((_OVR, _DIM), lambda t: (0, 0)),
        ],
        out_specs=pl.BlockSpec((_TB, _DIM), lambda t: (t, 0)),
        compiler_params=pltpu.CompilerParams(
            dimension_semantics=("parallel",),
        ),
    )(pos, ssum, dacc, b2, ywm, ywo)
    return out


# speedup vs baseline: 1.0279x; 1.0279x over previous
import jax
import jax.numpy as jnp
from jax.experimental import pallas as pl
from jax.experimental.pallas import tpu as pltpu

_DIM = 1024
_E = 8
_T = 2048
_F = 4 * _DIM
_CAP = 640
_MAINB = _E * _CAP
_B2 = 256
_OVG = 14
_OVR = _OVG * _B2
_GP = 64
_TB = 256

_f32 = jnp.float32
_bf16 = jnp.bfloat16


def _dotT(a, b):
    return jax.lax.dot_general(a, b, (((0,), (0,)), ((), ())),
                               preferred_element_type=_f32)


def _router_body(x_ref, wg_ref, bg_ref, pos_ref, sc_ref, ssum_ref, meta_ref):
    x = x_ref[...]
    logits = jnp.dot(x, wg_ref[...], preferred_element_type=_f32)
    logits = logits + bg_ref[...]
    m = jnp.max(logits, axis=-1, keepdims=True)
    p = jnp.exp(logits - m)
    scores = p / jnp.sum(p, axis=-1, keepdims=True)

    eio = jax.lax.broadcasted_iota(jnp.int32, (_T, _E), 1)
    m1 = jnp.max(scores, axis=-1, keepdims=True)
    i1 = jnp.min(jnp.where(scores >= m1, eio, _E), axis=-1, keepdims=True)
    masked = jnp.where(eio == i1, -jnp.inf, scores)
    m2 = jnp.max(masked, axis=-1, keepdims=True)
    i2 = jnp.min(jnp.where(masked >= m2, eio, _E), axis=-1, keepdims=True)

    a0 = (eio == i1).astype(_f32)
    a1 = (eio == i2).astype(_f32)

    ti = jax.lax.broadcasted_iota(jnp.int32, (_T, _T), 0)
    tj = jax.lax.broadcasted_iota(jnp.int32, (_T, _T), 1)
    ltri = (ti >= tj).astype(_bf16)
    a01 = jnp.concatenate([a0, a1], axis=1).astype(_bf16)
    cs = jnp.dot(ltri, a01, preferred_element_type=_f32)
    cs0 = cs[:, :_E]
    cs1 = cs[:, _E:]
    tot0 = cs0[_T - 1:_T, :]
    tot1 = cs1[_T - 1:_T, :]
    counts = (tot0 + tot1).astype(jnp.int32)

    ov = jnp.maximum(counts - _CAP, 0)
    ovblk = jnp.right_shift(ov + (_B2 - 1), 8)
    ei = jax.lax.broadcasted_iota(jnp.int32, (_E, _E), 0)
    ej = jax.lax.broadcasted_iota(jnp.int32, (_E, _E), 1)
    strict = (ei < ej).astype(_f32)
    ovex = jnp.dot(ovblk.astype(_f32), strict,
                   preferred_element_type=_f32)
    ovincl = ovex + ovblk.astype(_f32)
    ovstart = float(_MAINB) + ovex * float(_B2)

    ecap = (jax.lax.broadcasted_iota(jnp.int32, (1, _E), 1)
            .astype(_f32)) * float(_CAP)

    rank0 = jnp.sum(a0 * cs0, axis=1, keepdims=True) - 1.0
    rank1 = (jnp.sum(a1 * cs1, axis=1, keepdims=True) - 1.0
             + jnp.sum(a1 * tot0, axis=1, keepdims=True))
    st0m = jnp.sum(a0 * ecap, axis=1, keepdims=True)
    st1m = jnp.sum(a1 * ecap, axis=1, keepdims=True)
    st0o = jnp.sum(a0 * ovstart, axis=1, keepdims=True)
    st1o = jnp.sum(a1 * ovstart, axis=1, keepdims=True)
    pos0 = jnp.where(rank0 < float(_CAP), st0m + rank0,
                     st0o + rank0 - float(_CAP))
    pos1 = jnp.where(rank1 < float(_CAP), st1m + rank1,
                     st1o + rank1 - float(_CAP))
    pos_ref[...] = jnp.concatenate(
        [pos0, pos1], axis=1).astype(jnp.int32)
    sc_ref[...] = jnp.concatenate([m1, m2], axis=1)
    ssum_ref[...] = m1 + m2

    gi = jax.lax.broadcasted_iota(jnp.int32, (_GP, _E), 0)
    inb = jnp.broadcast_to(ovincl.astype(jnp.int32), (_GP, _E))
    beo = jnp.minimum(
        jnp.sum(jnp.where(gi >= inb, 1, 0), axis=1, keepdims=True), _E - 1)
    tot_blk = ovincl[0:1, _E - 1:_E].astype(jnp.int32)
    gj = jax.lax.broadcasted_iota(jnp.int32, (_GP, 1), 0)
    act = jnp.where(gj < tot_blk, 1, 0)
    meta_ref[...] = jnp.concatenate([beo, act], axis=1)


def _main_body(x_ref, pos_ref, sc_ref, w1_ref, b1_ref, w2_ref,
               yw_ref, d_ref):
    e = pl.program_id(0)

    @pl.when(e == 0)
    def _dz():
        d_ref[...] = jnp.zeros_like(d_ref)

    pos = pos_ref[...]
    liota = jax.lax.broadcasted_iota(jnp.int32, (_T, _CAP), 1) + e * _CAP
    m0 = (pos[:, 0:1] == liota).astype(_bf16)
    m1 = (pos[:, 1:2] == liota).astype(_bf16)
    mt = m0 + m1

    xg = _dotT(mt, x_ref[...]).astype(_bf16)

    fh = _F // 2
    y = jnp.zeros((_CAP, _DIM), _f32)
    d = jnp.zeros((1, _DIM), _f32)
    for f in range(2):
        w1c = w1_ref[0, :, f * fh:(f + 1) * fh]
        w2c = w2_ref[0, f * fh:(f + 1) * fh, :]
        b1c = b1_ref[0, 0, f * fh:(f + 1) * fh].reshape(1, fh)
        rbc = jnp.maximum(b1c, 0.0)
        hf = jnp.maximum(
            jnp.dot(xg, w1c, preferred_element_type=_f32) + b1c, 0.0) - rbc
        y = y + jnp.dot(hf.astype(_bf16), w2c, preferred_element_type=_f32)
        d = d + jnp.dot(rbc.astype(_bf16), w2c, preferred_element_type=_f32)

    d_ref[...] += d
    yw_ref[...] = y.astype(_bf16)


def _ov_body(beo_ref, act_ref, x_ref, pos_ref, sc_ref, b1_ref,
             w1_ref, w2_ref, yw_ref, w1buf, w2buf, w1sem, w2sem):
    g = pl.program_id(0)
    e = beo_ref[g]
    act = act_ref[g] == 1
    gp = jnp.maximum(g - 1, 0)
    same = jnp.logical_and(act_ref[gp] == 1, beo_ref[gp] == e)
    docopy = jnp.logical_and(act, jnp.logical_or(g == 0,
                                                 jnp.logical_not(same)))

    @pl.when(docopy)
    def _cp():
        c1 = pltpu.make_async_copy(w1_ref.at[e], w1buf, w1sem)
        c2 = pltpu.make_async_copy(w2_ref.at[e], w2buf, w2sem)
        c1.start()
        c2.start()
        c1.wait()
        c2.wait()

    @pl.when(jnp.logical_not(act))
    def _zero():
        yw_ref[...] = jnp.zeros_like(yw_ref)

    @pl.when(act)
    def _compute():
        ohe = (jax.lax.broadcasted_iota(jnp.int32, (1, _E), 1)
               == e).astype(_f32)
        b1v = jnp.dot(ohe, b1_ref[...], preferred_element_type=_f32)
        rb = jnp.maximum(b1v, 0.0)
        pos = pos_ref[...]
        liota = (jax.lax.broadcasted_iota(jnp.int32, (_T, _B2), 1)
                 + _MAINB + g * _B2)
        m0 = (pos[:, 0:1] == liota).astype(_bf16)
        m1 = (pos[:, 1:2] == liota).astype(_bf16)
        mt = m0 + m1
        xg = _dotT(mt, x_ref[...]).astype(_bf16)
        h = jnp.maximum(
            jnp.dot(xg, w1buf[...], preferred_element_type=_f32)
            + b1v, 0.0) - rb
        y = jnp.dot(h.astype(_bf16), w2buf[...], preferred_element_type=_f32)
        yw_ref[...] = y.astype(_bf16)


def _combine_body(pos_ref, sc_ref, ssum_ref, d_ref, b2_ref, ywm_ref, ywo_ref,
                  out_ref):
    crow = d_ref[...] + jnp.sum(b2_ref[...], axis=0, keepdims=True)
    pos = pos_ref[...]
    sc = sc_ref[...]
    ci = jax.lax.broadcasted_iota(jnp.int32, (_TB, _MAINB), 1)
    mm = (jnp.where(pos[:, 0:1] == ci, sc[:, 0:1], 0.0)
          + jnp.where(pos[:, 1:2] == ci, sc[:, 1:2], 0.0)).astype(_bf16)
    co = jax.lax.broadcasted_iota(jnp.int32, (_TB, _OVR), 1) + _MAINB
    mo = (jnp.where(pos[:, 0:1] == co, sc[:, 0:1], 0.0)
          + jnp.where(pos[:, 1:2] == co, sc[:, 1:2], 0.0)).astype(_bf16)
    out_ref[...] = (ssum_ref[...] * crow
                    + jnp.dot(mm, ywm_ref[...], preferred_element_type=_f32)
                    + jnp.dot(mo, ywo_ref[...], preferred_element_type=_f32))


def kernel(x, w_g, b_g, W1, b1, W2, b2):
    pos, sc, ssum, meta = pl.pallas_call(
        _router_body,
        out_shape=(
            jax.ShapeDtypeStruct((_T, 2), jnp.int32),
            jax.ShapeDtypeStruct((_T, 2), _f32),
            jax.ShapeDtypeStruct((_T, 1), _f32),
            jax.ShapeDtypeStruct((_GP, 2), jnp.int32),
        ),
        in_specs=[
            pl.BlockSpec((_T, _DIM), lambda: (0, 0)),
            pl.BlockSpec((_DIM, _E), lambda: (0, 0)),
            pl.BlockSpec((1, _E), lambda: (0, 0)),
        ],
        out_specs=(
            pl.BlockSpec((_T, 2), lambda: (0, 0)),
            pl.BlockSpec((_T, 2), lambda: (0, 0)),
            pl.BlockSpec((_T, 1), lambda: (0, 0)),
            pl.BlockSpec((_GP, 2), lambda: (0, 0)),
        ),
    )(x, w_g, b_g.reshape(1, _E))

    beo = meta[:_OVG, 0]
    act = meta[:_OVG, 1]
    xb = x.astype(_bf16)
    w1b = W1.astype(_bf16)
    w2b = W2.astype(_bf16)

    ywm, dacc = pl.pallas_call(
        _main_body,
        grid=(_E,),
        out_shape=(
            jax.ShapeDtypeStruct((_MAINB, _DIM), _bf16),
            jax.ShapeDtypeStruct((1, _DIM), _f32),
        ),
        in_specs=[
            pl.BlockSpec((_T, _DIM), lambda e: (0, 0)),
            pl.BlockSpec((_T, 2), lambda e: (0, 0)),
            pl.BlockSpec((_T, 2), lambda e: (0, 0)),
            pl.BlockSpec((1, _DIM, _F), lambda e: (e, 0, 0)),
            pl.BlockSpec((1, 1, _F), lambda e: (e, 0, 0)),
            pl.BlockSpec((1, _F, _DIM), lambda e: (e, 0, 0)),
        ],
        out_specs=(
            pl.BlockSpec((_CAP, _DIM), lambda e: (e, 0)),
            pl.BlockSpec((1, _DIM), lambda e: (0, 0)),
        ),
        compiler_params=pltpu.CompilerParams(
            dimension_semantics=("arbitrary",),
        ),
    )(xb, pos, sc, w1b, b1.reshape(_E, 1, _F), w2b)

    ov_spec = pltpu.PrefetchScalarGridSpec(
        num_scalar_prefetch=2,
        grid=(_OVG,),
        in_specs=[
            pl.BlockSpec((_T, _DIM), lambda g, beo, act: (0, 0)),
            pl.BlockSpec((_T, 2), lambda g, beo, act: (0, 0)),
            pl.BlockSpec((_T, 2), lambda g, beo, act: (0, 0)),
            pl.BlockSpec((_E, _F), lambda g, beo, act: (0, 0)),
            pl.BlockSpec(memory_space=pltpu.HBM),
            pl.BlockSpec(memory_space=pltpu.HBM),
        ],
        out_specs=pl.BlockSpec((_B2, _DIM), lambda g, beo, act: (g, 0)),
        scratch_shapes=[
            pltpu.VMEM((_DIM, _F), _bf16),
            pltpu.VMEM((_F, _DIM), _bf16),
            pltpu.SemaphoreType.DMA,
            pltpu.SemaphoreType.DMA,
        ],
    )
    ywo = pl.pallas_call(
        _ov_body,
        grid_spec=ov_spec,
        out_shape=jax.ShapeDtypeStruct((_OVR, _DIM), _bf16),
        compiler_params=pltpu.CompilerParams(
            dimension_semantics=("arbitrary",),
        ),
    )(beo, act, xb, pos, sc, b1, w1b, w2b)

    out = pl.pallas_call(
        _combine_body,
        grid=(_T // _TB,),
        out_shape=jax.ShapeDtypeStruct((_T, _DIM), _f32),
        in_specs=[
            pl.BlockSpec((_TB, 2), lambda t: (t, 0)),
            pl.BlockSpec((_TB, 2), lambda t: (t, 0)),
            pl.BlockSpec((_TB, 1), lambda t: (t, 0)),
            pl.BlockSpec((1, _DIM), lambda t: (0, 0)),
            pl.BlockSpec((_E, _DIM), lambda t: (0, 0)),
            pl.BlockSpec((_MAINB, _DIM), lambda t: (0, 0)),
            pl.BlockSpec((_OVR, _DIM), lambda t: (0, 0)),
        ],
        out_specs=pl.BlockSpec((_TB, _DIM), lambda t: (t, 0)),
        compiler_params=pltpu.CompilerParams(
            dimension_semantics=("parallel",),
        ),
    )(pos, sc, ssum, dacc, b2, ywm, ywo)
    return out
